# baseline (speedup 1.0000x reference)
import functools

import jax
import jax.numpy as jnp
from jax import lax
from jax.experimental import pallas as pl
from jax.experimental.pallas import tpu as pltpu
from jax.experimental.pallas import tpu_sc as plsc

_LANES = 16
_NBINS = 10001
_NB_PAD = 10240
_CHUNK = 4096
_NBUF = 2
_UNROLL = 16


def _phase1(preds, targets):
    n = preds.shape[0]
    info = plsc.get_sparse_core_info()
    nc, ns = info.num_cores, info.num_subcores
    nw = nc * ns
    per_tile = n // nw
    nchunks = per_tile // _CHUNK
    mesh = plsc.VectorSubcoreMesh(core_axis_name="c", subcore_axis_name="s")

    @functools.partial(
        pl.kernel,
        out_type=jax.ShapeDtypeStruct((nw, 2 * _NB_PAD), jnp.float32),
        mesh=mesh,
        compiler_params=pltpu.CompilerParams(needs_layout_passes=False),
        scratch_types=(
            [pltpu.VMEM((_CHUNK,), jnp.float32) for _ in range(2 * _NBUF)]
            + [pltpu.VMEM((2 * _NB_PAD,), jnp.float32)]
            + [pltpu.SemaphoreType.DMA for _ in range(2 * _NBUF)]
        ),
    )
    def k(preds_hbm, targets_hbm, out_hbm, *rest):
        pbufs = rest[:_NBUF]
        tbufs = rest[_NBUF:2 * _NBUF]
        hist = rest[2 * _NBUF]
        psems = rest[2 * _NBUF + 1:3 * _NBUF + 1]
        tsems = rest[3 * _NBUF + 1:4 * _NBUF + 1]

        cid = lax.axis_index("c")
        sid = lax.axis_index("s")
        wid = sid * nc + cid

        zeros = jnp.zeros((_LANES,), jnp.float32)
        ones = jnp.ones((_LANES,), jnp.float32)

        base = wid * per_tile

        for b in range(_NBUF):
            off = base + b * _CHUNK
            pltpu.async_copy(preds_hbm.at[pl.ds(off, _CHUNK)], pbufs[b], psems[b])
            pltpu.async_copy(targets_hbm.at[pl.ds(off, _CHUNK)], tbufs[b], tsems[b])

        @plsc.parallel_loop(0, 2 * _NB_PAD // _LANES, unroll=8)
        def _(i):
            hist[pl.ds(i * _LANES, _LANES)] = zeros

        def cbody(jj, _):
            for b in range(_NBUF):
                j = jj * _NBUF + b
                pb, tb = pbufs[b], tbufs[b]
                pltpu.make_async_copy(
                    preds_hbm.at[pl.ds(0, _CHUNK)], pb, psems[b]).wait()
                pltpu.make_async_copy(
                    targets_hbm.at[pl.ds(0, _CHUNK)], tb, tsems[b]).wait()

                @plsc.parallel_loop(0, _CHUNK // _LANES, unroll=_UNROLL)
                def _(i, pb=pb, tb=tb):
                    ds = pl.ds(i * _LANES, _LANES)
                    p = pb[ds]
                    t = tb[ds]
                    s = 1.0 / (1.0 + jnp.exp(-p))
                    bn = (10000.0 * s).astype(jnp.int32)
                    half = jnp.where(t >= 0.5, _NB_PAD, 0)
                    plsc.addupdate_scatter(hist, [bn + half], ones)

                nxt = j + _NBUF

                @pl.when(nxt < nchunks)
                def _(b=b, pb=pb, tb=tb, nxt=nxt):
                    off = base + nxt * _CHUNK
                    pltpu.async_copy(
                        preds_hbm.at[pl.ds(off, _CHUNK)], pb, psems[b])
                    pltpu.async_copy(
                        targets_hbm.at[pl.ds(off, _CHUNK)], tb, tsems[b])
            return 0

        lax.fori_loop(0, nchunks // _NBUF, cbody, 0)

        pltpu.sync_copy(hist, out_hbm.at[wid])

    return k(preds, targets)


def _phase2_tc(part4):
    nc = part4.shape[0]
    r, c = part4.shape[2], part4.shape[3]

    def _scan0(x, n):
        sh = 1
        while sh < n:
            z = jnp.zeros((sh,) + x.shape[1:], jnp.float32)
            x = x + jnp.concatenate([z, x[: n - sh]], axis=0)
            sh *= 2
        return x

    def _scan1(x, n):
        sh = 1
        while sh < n:
            z = jnp.zeros(x.shape[:1] + (sh,), jnp.float32)
            x = x + jnp.concatenate([z, x[:, : n - sh]], axis=1)
            sh *= 2
        return x

    def cum2d(x):
        rowcum = _scan1(x, c)
        rowtot = rowcum[:, c - 1:c]
        rowpref = _scan0(rowtot, r) - rowtot
        return rowcum + rowpref

    def body(p_ref, o_ref):
        fp = p_ref[0, 0]
        tp = p_ref[0, 1]
        for k in range(1, nc):
            fp = fp + p_ref[k, 0]
            tp = tp + p_ref[k, 1]
        cum_tp = cum2d(tp)
        cum_fp = cum2d(fp)
        tot_tp = jnp.sum(tp)
        tot_fp = jnp.sum(fp)
        d = jnp.abs(cum_tp / tot_tp - cum_fp / tot_fp)
        o_ref[...] = jnp.broadcast_to(jnp.max(d), (1, 1))

    return pl.pallas_call(
        body,
        out_shape=jax.ShapeDtypeStruct((1, 1), jnp.float32),
    )(part4)


def kernel(preds, targets):
    part = _phase1(preds, targets)
    ks = _phase2_tc(part.reshape(part.shape[0], 2, 80, 128))
    return ks[0, 0]


# speedup vs baseline: 1.0547x; 1.0547x over previous
import functools

import jax
import jax.numpy as jnp
from jax import lax
from jax.experimental import pallas as pl
from jax.experimental.pallas import tpu as pltpu
from jax.experimental.pallas import tpu_sc as plsc

_LANES = 16
_NBINS = 10001
_NB_PAD = 10240
_CHUNK = 8192
_NBUF = 2
_UNROLL = 16


def _phase1(preds, targets):
    n = preds.shape[0]
    info = plsc.get_sparse_core_info()
    nc, ns = info.num_cores, info.num_subcores
    nw = nc * ns
    per_tile = n // nw
    nchunks = per_tile // _CHUNK
    mesh = plsc.VectorSubcoreMesh(core_axis_name="c", subcore_axis_name="s")

    @functools.partial(
        pl.kernel,
        out_type=jax.ShapeDtypeStruct((nw, 2 * _NB_PAD), jnp.float32),
        mesh=mesh,
        compiler_params=pltpu.CompilerParams(needs_layout_passes=False),
        scratch_types=(
            [pltpu.VMEM((_CHUNK,), jnp.float32) for _ in range(2 * _NBUF)]
            + [pltpu.VMEM((2 * _NB_PAD,), jnp.float32)]
            + [pltpu.SemaphoreType.DMA for _ in range(2 * _NBUF)]
        ),
    )
    def k(preds_hbm, targets_hbm, out_hbm, *rest):
        pbufs = rest[:_NBUF]
        tbufs = rest[_NBUF:2 * _NBUF]
        hist = rest[2 * _NBUF]
        psems = rest[2 * _NBUF + 1:3 * _NBUF + 1]
        tsems = rest[3 * _NBUF + 1:4 * _NBUF + 1]

        cid = lax.axis_index("c")
        sid = lax.axis_index("s")
        wid = sid * nc + cid

        zeros = jnp.zeros((_LANES,), jnp.float32)
        ones = jnp.ones((_LANES,), jnp.float32)

        base = wid * per_tile

        for b in range(_NBUF):
            off = base + b * _CHUNK
            pltpu.async_copy(preds_hbm.at[pl.ds(off, _CHUNK)], pbufs[b], psems[b])
            pltpu.async_copy(targets_hbm.at[pl.ds(off, _CHUNK)], tbufs[b], tsems[b])

        @plsc.parallel_loop(0, 2 * _NB_PAD // _LANES, unroll=8)
        def _(i):
            hist[pl.ds(i * _LANES, _LANES)] = zeros

        def cbody(jj, _):
            for b in range(_NBUF):
                j = jj * _NBUF + b
                pb, tb = pbufs[b], tbufs[b]
                pltpu.make_async_copy(
                    preds_hbm.at[pl.ds(0, _CHUNK)], pb, psems[b]).wait()
                pltpu.make_async_copy(
                    targets_hbm.at[pl.ds(0, _CHUNK)], tb, tsems[b]).wait()

                @plsc.parallel_loop(0, _CHUNK // _LANES, unroll=_UNROLL)
                def _(i, pb=pb, tb=tb):
                    ds = pl.ds(i * _LANES, _LANES)
                    p = pb[ds]
                    t = tb[ds]
                    s = 1.0 / (1.0 + jnp.exp(-p))
                    bn = (10000.0 * s).astype(jnp.int32)
                    half = jnp.where(t >= 0.5, _NB_PAD, 0)
                    plsc.addupdate_scatter(hist, [bn + half], ones)

                nxt = j + _NBUF

                @pl.when(nxt < nchunks)
                def _(b=b, pb=pb, tb=tb, nxt=nxt):
                    off = base + nxt * _CHUNK
                    pltpu.async_copy(
                        preds_hbm.at[pl.ds(off, _CHUNK)], pb, psems[b])
                    pltpu.async_copy(
                        targets_hbm.at[pl.ds(off, _CHUNK)], tb, tsems[b])
            return 0

        lax.fori_loop(0, nchunks // _NBUF, cbody, 0)

        pltpu.sync_copy(hist, out_hbm.at[wid])

    return k(preds, targets)


def _phase2_tc(part4):
    nc = part4.shape[0]
    r, c = part4.shape[2], part4.shape[3]

    def _scan0(x, n):
        sh = 1
        while sh < n:
            z = jnp.zeros((sh,) + x.shape[1:], jnp.float32)
            x = x + jnp.concatenate([z, x[: n - sh]], axis=0)
            sh *= 2
        return x

    def _scan1(x, n):
        sh = 1
        while sh < n:
            z = jnp.zeros(x.shape[:1] + (sh,), jnp.float32)
            x = x + jnp.concatenate([z, x[:, : n - sh]], axis=1)
            sh *= 2
        return x

    def cum2d(x):
        rowcum = _scan1(x, c)
        rowtot = rowcum[:, c - 1:c]
        rowpref = _scan0(rowtot, r) - rowtot
        return rowcum + rowpref

    def body(p_ref, o_ref):
        fp = p_ref[0, 0]
        tp = p_ref[0, 1]
        for k in range(1, nc):
            fp = fp + p_ref[k, 0]
            tp = tp + p_ref[k, 1]
        cum_tp = cum2d(tp)
        cum_fp = cum2d(fp)
        tot_tp = jnp.sum(tp)
        tot_fp = jnp.sum(fp)
        d = jnp.abs(cum_tp / tot_tp - cum_fp / tot_fp)
        o_ref[...] = jnp.broadcast_to(jnp.max(d), (1, 1))

    return pl.pallas_call(
        body,
        out_shape=jax.ShapeDtypeStruct((1, 1), jnp.float32),
    )(part4)


def kernel(preds, targets):
    part = _phase1(preds, targets)
    ks = _phase2_tc(part.reshape(part.shape[0], 2, 80, 128))
    return ks[0, 0]
